# baseline (speedup 1.0000x reference)
import numpy as np
import jax
import jax.numpy as jnp
from jax.experimental import pallas as pl
from jax.experimental.pallas import tpu as pltpu

_SIGMA = (1.0, 1.0, 1.0)


def _matrices_kernel(v_ref, az_ref, ay_ref, ax_ref):
    def mk(n, scale):
        i = jax.lax.broadcasted_iota(jnp.int32, (n, n), 0)
        j = jax.lax.broadcasted_iota(jnp.int32, (n, n), 1)
        d = (i - j).astype(jnp.float32) * scale
        k = jnp.exp(-0.5 * d * d)
        return k / jnp.sum(k, axis=1, keepdims=True)

    d = az_ref.shape[0]
    h = ay_ref.shape[0]
    w = ax_ref.shape[0]
    az_ref[...] = mk(d, v_ref[0] / _SIGMA[0])
    ay_ref[...] = mk(h, v_ref[2] / _SIGMA[2])
    ax_ref[...] = mk(w, v_ref[1] / _SIGMA[1])


def _zpass_kernel(az_ref, x_ref, o_ref):
    o_ref[0] = jnp.dot(az_ref[...], x_ref[0],
                       preferred_element_type=jnp.float32)


def _yxpass_kernel(ay_ref, ax_ref, x_ref, o_ref):
    t = jnp.dot(ay_ref[...], x_ref[0], preferred_element_type=jnp.float32)
    o_ref[0] = jax.lax.dot_general(
        t, ax_ref[...], (((1,), (1,)), ((), ())),
        preferred_element_type=jnp.float32)


@jax.jit
def kernel(input_, image, v_gamma):
    c, d, h, w = input_.shape

    az, ay, ax = pl.pallas_call(
        _matrices_kernel,
        in_specs=[pl.BlockSpec(memory_space=pltpu.SMEM)],
        out_shape=[
            jax.ShapeDtypeStruct((d, d), jnp.float32),
            jax.ShapeDtypeStruct((h, h), jnp.float32),
            jax.ShapeDtypeStruct((w, w), jnp.float32),
        ],
    )(v_gamma)

    s = h * w
    sb = 2048
    q = input_.reshape(c, d, s)
    t1 = pl.pallas_call(
        _zpass_kernel,
        grid=(c, s // sb),
        in_specs=[
            pl.BlockSpec((d, d), lambda ci, si: (0, 0)),
            pl.BlockSpec((1, d, sb), lambda ci, si: (ci, 0, si)),
        ],
        out_specs=pl.BlockSpec((1, d, sb), lambda ci, si: (ci, 0, si)),
        out_shape=jax.ShapeDtypeStruct((c, d, s), jnp.float32),
    )(az, q)

    t1 = t1.reshape(c * d, h, w)
    out = pl.pallas_call(
        _yxpass_kernel,
        grid=(c * d,),
        in_specs=[
            pl.BlockSpec((h, h), lambda b: (0, 0)),
            pl.BlockSpec((w, w), lambda b: (0, 0)),
            pl.BlockSpec((1, h, w), lambda b: (b, 0, 0)),
        ],
        out_specs=pl.BlockSpec((1, h, w), lambda b: (b, 0, 0)),
        out_shape=jax.ShapeDtypeStruct((c * d, h, w), jnp.float32),
    )(ay, ax, t1)
    return out.reshape(c, d, h, w)


# speedup vs baseline: 2.2127x; 2.2127x over previous
import functools

import jax
import jax.numpy as jnp
from jax.experimental import pallas as pl
from jax.experimental.pallas import tpu as pltpu

_SIGMA = (1.0, 1.0, 1.0)


def _gauss_matrix(n, scale):
    i = jax.lax.broadcasted_iota(jnp.int32, (n, n), 0)
    j = jax.lax.broadcasted_iota(jnp.int32, (n, n), 1)
    d = (i - j).astype(jnp.float32) * scale
    k = jnp.exp(-0.5 * d * d)
    return k / jnp.sum(k, axis=1, keepdims=True)


def _zpass_kernel(v_ref, x_ref, o_ref):
    d = x_ref.shape[0]
    az = _gauss_matrix(d, v_ref[0] / _SIGMA[0])
    o_ref[...] = jnp.dot(az, x_ref[...], preferred_element_type=jnp.float32)


def _yxpass_kernel(v_ref, x_ref, o_ref, *, planes, h, w):
    ay = _gauss_matrix(h, v_ref[2] / _SIGMA[2])
    ax = _gauss_matrix(w, v_ref[1] / _SIGMA[1])
    p = jax.lax.dot_general(
        x_ref[...], ax, (((1,), (1,)), ((), ())),
        preferred_element_type=jnp.float32)
    for i in range(planes):
        o_ref[i * h:(i + 1) * h, :] = jnp.dot(
            ay, p[i * h:(i + 1) * h, :], preferred_element_type=jnp.float32)


@jax.jit
def kernel(input_, image, v_gamma):
    c, d, h, w = input_.shape
    s = h * w

    q = input_.reshape(c * d, s)
    t1 = pl.pallas_call(
        _zpass_kernel,
        grid=(c,),
        in_specs=[
            pl.BlockSpec(memory_space=pltpu.SMEM),
            pl.BlockSpec((d, s), lambda ci: (ci, 0)),
        ],
        out_specs=pl.BlockSpec((d, s), lambda ci: (ci, 0)),
        out_shape=jax.ShapeDtypeStruct((c * d, s), jnp.float32),
    )(v_gamma, q)

    planes = 8
    t1 = t1.reshape(c * d * h, w)
    body = functools.partial(_yxpass_kernel, planes=planes, h=h, w=w)
    out = pl.pallas_call(
        body,
        grid=(c * d // planes,),
        in_specs=[
            pl.BlockSpec(memory_space=pltpu.SMEM),
            pl.BlockSpec((planes * h, w), lambda b: (b, 0)),
        ],
        out_specs=pl.BlockSpec((planes * h, w), lambda b: (b, 0)),
        out_shape=jax.ShapeDtypeStruct((c * d * h, w), jnp.float32),
    )(v_gamma, t1)
    return out.reshape(c, d, h, w)


# speedup vs baseline: 4.5137x; 2.0400x over previous
import jax
import jax.numpy as jnp
from jax.experimental import pallas as pl
from jax.experimental.pallas import tpu as pltpu

_SIGMA = (1.0, 1.0, 1.0)
_ZRAD = 6


def _gauss_matrix(n, scale):
    i = jax.lax.broadcasted_iota(jnp.int32, (n, n), 0)
    j = jax.lax.broadcasted_iota(jnp.int32, (n, n), 1)
    d = (i - j).astype(jnp.float32) * scale
    k = jnp.exp(-0.5 * d * d)
    return k / jnp.sum(k, axis=1, keepdims=True)


def _fused_kernel(v_ref, x_ref, o_ref):
    d = x_ref.shape[1]
    h = x_ref.shape[2]
    w = x_ref.shape[3]
    ay = _gauss_matrix(h, v_ref[2] / _SIGMA[2])
    ax = _gauss_matrix(w, v_ref[1] / _SIGMA[1])

    zscale = v_ref[0] / _SIGMA[0]
    ki = jax.lax.broadcasted_iota(jnp.int32, (1, _ZRAD + 1), 1)
    kd = ki.astype(jnp.float32) * zscale
    taps = jnp.exp(-0.5 * kd * kd)
    zi = jax.lax.broadcasted_iota(jnp.int32, (d, d), 0)
    zj = jax.lax.broadcasted_iota(jnp.int32, (d, d), 1)
    zd = (zi - zj).astype(jnp.float32) * zscale
    inv_sz = 1.0 / jnp.sum(jnp.exp(-0.5 * zd * zd), axis=1, keepdims=True)

    o_ref[...] = jnp.zeros_like(o_ref)
    for di in range(d):
        t = jax.lax.dot_general(
            x_ref[0, di], ax, (((1,), (1,)), ((), ())),
            preferred_element_type=jnp.float32)
        p = jnp.dot(ay, t, preferred_element_type=jnp.float32)
        for k in range(-_ZRAD, _ZRAD + 1):
            zo = di + k
            if 0 <= zo < d:
                g = jax.lax.broadcast_in_dim(
                    taps[0:1, abs(k):abs(k) + 1], (h, w), (0, 1))
                o_ref[0, zo] += g * p
    for di in range(d):
        s = jax.lax.broadcast_in_dim(
            inv_sz[di:di + 1, 0:1], (h, w), (0, 1))
        o_ref[0, di] *= s


@jax.jit
def kernel(input_, image, v_gamma):
    c, d, h, w = input_.shape
    return pl.pallas_call(
        _fused_kernel,
        grid=(c,),
        in_specs=[
            pl.BlockSpec(memory_space=pltpu.SMEM),
            pl.BlockSpec((1, d, h, w), lambda ci: (ci, 0, 0, 0)),
        ],
        out_specs=pl.BlockSpec((1, d, h, w), lambda ci: (ci, 0, 0, 0)),
        out_shape=jax.ShapeDtypeStruct((c, d, h, w), jnp.float32),
    )(v_gamma, input_)


# speedup vs baseline: 4.8102x; 1.0657x over previous
import jax
import jax.numpy as jnp
from jax.experimental import pallas as pl
from jax.experimental.pallas import tpu as pltpu

_SIGMA = (1.0, 1.0, 1.0)
_T = 8


def _gauss_matrix(n, scale):
    i = jax.lax.broadcasted_iota(jnp.int32, (n, n), 0)
    j = jax.lax.broadcasted_iota(jnp.int32, (n, n), 1)
    d = (i - j).astype(jnp.float32) * scale
    k = jnp.exp(-0.5 * d * d)
    return k / jnp.sum(k, axis=1, keepdims=True)


def _kron_gauss_eye(d, scale):
    n = d * _T
    a = jax.lax.broadcasted_iota(jnp.int32, (n, n), 0)
    b = jax.lax.broadcasted_iota(jnp.int32, (n, n), 1)
    dz = ((a // _T) - (b // _T)).astype(jnp.float32) * scale
    k = jnp.exp(-0.5 * dz * dz)
    k = jnp.where((a % _T) == (b % _T), k, 0.0)
    return k / jnp.sum(k, axis=1, keepdims=True)


def _fused_kernel(v_ref, x_ref, o_ref, p_ref):
    d, h, w = x_ref.shape[1], x_ref.shape[2], x_ref.shape[3]
    ay = _gauss_matrix(h, v_ref[2] / _SIGMA[2])
    ax = _gauss_matrix(w, v_ref[1] / _SIGMA[1])
    azk = _kron_gauss_eye(d, v_ref[0] / _SIGMA[0])

    for di in range(d):
        t = jax.lax.dot_general(
            x_ref[0, di], ax, (((1,), (1,)), ((), ())),
            preferred_element_type=jnp.float32)
        p_ref[di] = jnp.dot(ay, t, preferred_element_type=jnp.float32)

    for hb in range(h // _T):
        blk = p_ref[:, hb * _T:(hb + 1) * _T, :]
        ob = jnp.dot(azk, blk.reshape(d * _T, w),
                     preferred_element_type=jnp.float32)
        o_ref[0, :, hb * _T:(hb + 1) * _T, :] = ob.reshape(d, _T, w)


@jax.jit
def kernel(input_, image, v_gamma):
    c, d, h, w = input_.shape
    return pl.pallas_call(
        _fused_kernel,
        grid=(c,),
        in_specs=[
            pl.BlockSpec(memory_space=pltpu.SMEM),
            pl.BlockSpec((1, d, h, w), lambda ci: (ci, 0, 0, 0)),
        ],
        out_specs=pl.BlockSpec((1, d, h, w), lambda ci: (ci, 0, 0, 0)),
        out_shape=jax.ShapeDtypeStruct((c, d, h, w), jnp.float32),
        scratch_shapes=[pltpu.VMEM((d, h, w), jnp.float32)],
    )(v_gamma, input_)


# speedup vs baseline: 12.7874x; 2.6584x over previous
import jax
import jax.numpy as jnp
from jax.experimental import pallas as pl
from jax.experimental.pallas import tpu as pltpu

_SIGMA = (1.0, 1.0, 1.0)
_T = 8


def _gauss_matrix(n, scale):
    i = jax.lax.broadcasted_iota(jnp.int32, (n, n), 0)
    j = jax.lax.broadcasted_iota(jnp.int32, (n, n), 1)
    d = (i - j).astype(jnp.float32) * scale
    k = jnp.exp(-0.5 * d * d)
    return k / jnp.sum(k, axis=1, keepdims=True)


def _kron_gauss_eye(d, scale):
    n = d * _T
    a = jax.lax.broadcasted_iota(jnp.int32, (n, n), 0)
    b = jax.lax.broadcasted_iota(jnp.int32, (n, n), 1)
    dz = ((a // _T) - (b // _T)).astype(jnp.float32) * scale
    k = jnp.exp(-0.5 * dz * dz)
    k = jnp.where((a % _T) == (b % _T), k, 0.0)
    return k / jnp.sum(k, axis=1, keepdims=True)


def _fused_kernel(v_ref, x_ref, o_ref, p_ref):
    d, h, w = x_ref.shape[1], x_ref.shape[2], x_ref.shape[3]
    ay = _gauss_matrix(h, v_ref[2] / _SIGMA[2])
    ax = _gauss_matrix(w, v_ref[1] / _SIGMA[1])
    azk = _kron_gauss_eye(d, v_ref[0] / _SIGMA[0])

    t = jax.lax.dot_general(
        x_ref[0].reshape(d * h, w), ax, (((1,), (1,)), ((), ())),
        preferred_element_type=jnp.float32).reshape(d, h, w)
    for di in range(d):
        p_ref[di] = jnp.dot(ay, t[di], preferred_element_type=jnp.float32)

    for hb in range(h // _T):
        blk = p_ref[:, hb * _T:(hb + 1) * _T, :]
        ob = jnp.dot(azk, blk.reshape(d * _T, w),
                     preferred_element_type=jnp.float32)
        o_ref[0, :, hb * _T:(hb + 1) * _T, :] = ob.reshape(d, _T, w)


@jax.jit
def kernel(input_, image, v_gamma):
    c, d, h, w = input_.shape
    return pl.pallas_call(
        _fused_kernel,
        grid=(c,),
        in_specs=[
            pl.BlockSpec(memory_space=pltpu.SMEM),
            pl.BlockSpec((1, d, h, w), lambda ci: (ci, 0, 0, 0)),
        ],
        out_specs=pl.BlockSpec((1, d, h, w), lambda ci: (ci, 0, 0, 0)),
        out_shape=jax.ShapeDtypeStruct((c, d, h, w), jnp.float32),
        scratch_shapes=[pltpu.VMEM((d, h, w), jnp.float32)],
        compiler_params=pltpu.CompilerParams(
            dimension_semantics=("parallel",)),
    )(v_gamma, input_)
